# baseline (speedup 1.0000x reference)
import functools

import jax
import jax.numpy as jnp
from jax import lax
from jax.experimental import pallas as pl
from jax.experimental.pallas import tpu as pltpu
from jax.experimental.pallas import tpu_sc as plsc

NOP = 100000
HIDDEN_DIM = 64
BATCH = 16384

_info = plsc.get_sparse_core_info()
_NC, _NS = _info.num_cores, _info.num_subcores
_NW = _NC * _NS
_ROWS_PER_W = HIDDEN_DIM // _NW

_AUX0 = NOP - 128
_S = (33408, 33408, 33152)
_LO = (0, 33408, 66816)
_ALIGNED_END = 99968
_SLABW = 33408
_CHUNK = 2048
_NCH = BATCH // _CHUNK


def _make_gather():
    mesh = plsc.VectorSubcoreMesh(core_axis_name="c", subcore_axis_name="s")

    @functools.partial(
        pl.kernel,
        mesh=mesh,
        out_type=jax.ShapeDtypeStruct((HIDDEN_DIM, BATCH), jnp.float32),
        scratch_types=[
            pltpu.VMEM((_CHUNK,), jnp.int32),
            pltpu.VMEM((_CHUNK,), jnp.int32),
            pltpu.VMEM((1, _SLABW), jnp.float32),
            pltpu.VMEM((1, _SLABW), jnp.float32),
            pltpu.VMEM((1, BATCH), jnp.float32),
            pltpu.SemaphoreType.DMA,
            pltpu.SemaphoreType.DMA,
            pltpu.SemaphoreType.DMA,
            pltpu.SemaphoreType.DMA,
        ],
        compiler_params=pltpu.CompilerParams(
            needs_layout_passes=False, skip_device_barrier=True
        ),
    )
    def gather_kernel(
        idx_hbm, tT_hbm, tailT_hbm, outT_hbm,
        ix0, ix1, sl0, sl1, orow_v,
        isem0, isem1, ssem0, ssem1,
    ):
        wid = lax.axis_index("s") * _NC + lax.axis_index("c")
        lane = lax.iota(jnp.int32, 16)
        zero16 = jnp.zeros((16,), jnp.int32)
        ix = (ix0, ix1)
        sl = (sl0, sl1)
        isem = (isem0, isem1)
        ssem = (ssem0, ssem1)

        def issue_slab(g):
            hp, s = divmod(g, 3)
            h = _ROWS_PER_W * wid + hp
            buf = sl[g % 2]
            pltpu.async_copy(
                tT_hbm.at[pl.ds(h, 1), pl.ds(_LO[s], _S[s])],
                buf.at[:, pl.ds(0, _S[s])],
                ssem[g % 2],
            )
            if s == 2:
                pltpu.async_copy(
                    tailT_hbm.at[pl.ds(h, 1), :],
                    buf.at[:, pl.ds(_S[2], 128)],
                    ssem[g % 2],
                )

        def wait_slab(g):
            _, s = divmod(g, 3)
            buf = sl[g % 2]
            pltpu.make_async_copy(
                tT_hbm.at[pl.ds(0, 1), pl.ds(0, _S[s])],
                buf.at[:, pl.ds(0, _S[s])],
                ssem[g % 2],
            ).wait()
            if s == 2:
                pltpu.make_async_copy(
                    tailT_hbm.at[pl.ds(0, 1), :],
                    buf.at[:, pl.ds(_S[2], 128)],
                    ssem[g % 2],
                ).wait()

        def issue_idx(c, p):
            pltpu.async_copy(
                idx_hbm.at[pl.ds(c * _CHUNK, _CHUNK)], ix[p], isem[p]
            )

        def wait_idx(p):
            pltpu.make_async_copy(
                idx_hbm.at[pl.ds(0, _CHUNK)], ix[p], isem[p]
            ).wait()

        issue_slab(0)
        issue_idx(0, 0)
        issue_idx(1, 1)

        for g in range(2 * 3):
            _, s = divmod(g, 3)
            slab = sl[g % 2]
            wait_slab(g)
            if g + 1 < 2 * 3:
                issue_slab(g + 1)

            def scan(cb, p, s=s, slab=slab):
                base = cb * _CHUNK

                @plsc.parallel_loop(0, _CHUNK, step=16, unroll=8)
                def _(k):
                    iv = ix[p][pl.ds(k, 16)]
                    if s == 0:
                        m = iv < _S[0]
                        rel = iv
                    elif s == 1:
                        m = (iv >= _LO[1]) & (iv < _LO[2])
                        rel = iv - _LO[1]
                    else:
                        m = iv >= _LO[2]
                        rel = jnp.where(
                            iv >= _AUX0,
                            iv - (_AUX0 - _S[2]),
                            iv - _LO[2],
                        )
                    gv = plsc.load_gather(slab, [zero16, rel], mask=m)
                    plsc.store_scatter(
                        orow_v, [zero16, lane + (base + k)], gv, mask=m
                    )

            def chunk_pair(c2, _, s=s, slab=slab):
                cb = c2 * 2
                wait_idx(0)
                scan(cb, 0)
                issue_idx((cb + 2) % _NCH, 0)
                wait_idx(1)
                scan(cb + 1, 1)
                issue_idx((cb + 3) % _NCH, 1)
                return ()

            lax.fori_loop(0, _NCH // 2, chunk_pair, ())

            if s == 2:
                hp = g // 3
                h = _ROWS_PER_W * wid + hp
                pltpu.sync_copy(orow_v, outT_hbm.at[pl.ds(h, 1), :])

        wait_idx(0)
        wait_idx(1)

    return gather_kernel


_gather = _make_gather()


def kernel(problem_id, embedding_table):
    tail_t = embedding_table[_AUX0:].T
    out_t = _gather(problem_id, embedding_table.T, tail_t)
    return out_t.T


# speedup vs baseline: 1.3888x; 1.3888x over previous
import functools

import jax
import jax.numpy as jnp
from jax import lax
from jax.experimental import pallas as pl
from jax.experimental.pallas import tpu as pltpu
from jax.experimental.pallas import tpu_sc as plsc

NOP = 100000
HIDDEN_DIM = 64
BATCH = 16384

_info = plsc.get_sparse_core_info()
_NC, _NS = _info.num_cores, _info.num_subcores
_NW = _NC * _NS
_ROWS_PER_W = HIDDEN_DIM // _NW
_SPLIT = 50048
_ALIGNED_END = 99968
_AUX0 = NOP - 128
_LN1 = _ALIGNED_END - _SPLIT


def _make_gather():
    mesh = plsc.VectorSubcoreMesh(core_axis_name="c", subcore_axis_name="s")

    @functools.partial(
        pl.kernel,
        mesh=mesh,
        out_type=jax.ShapeDtypeStruct((HIDDEN_DIM, BATCH), jnp.float32),
        scratch_types=[
            pltpu.VMEM((BATCH,), jnp.int32),
            pltpu.VMEM((1, _SPLIT), jnp.float32),
            pltpu.VMEM((1, BATCH), jnp.float32),
            pltpu.SemaphoreType.DMA,
        ],
        compiler_params=pltpu.CompilerParams(
            needs_layout_passes=False, skip_device_barrier=True
        ),
    )
    def gather_kernel(
        idx_hbm, tT_hbm, tailT_hbm, outT_hbm, idx_v, slab_v, orow_v, sem
    ):
        wid = lax.axis_index("s") * _NC + lax.axis_index("c")
        lane = lax.iota(jnp.int32, 16)
        zero16 = jnp.zeros((16,), jnp.int32)

        def slab_dma(h, c0, ln):
            copies = []
            off = 0
            while off < ln:
                cl = min(12544, ln - off)
                copies.append(
                    pltpu.async_copy(
                        tT_hbm.at[pl.ds(h, 1), pl.ds(c0 + off, cl)],
                        slab_v.at[:, pl.ds(off, cl)],
                        sem,
                    )
                )
                off += cl
            return copies

        for hp in range(_ROWS_PER_W):
            h = _ROWS_PER_W * wid + hp

            copies = slab_dma(h, 0, _SPLIT)
            if hp == 0:
                pltpu.sync_copy(idx_hbm, idx_v)
            for c in copies:
                c.wait()

            @plsc.parallel_loop(0, BATCH, step=16, unroll=8)
            def step0(k):
                iv = idx_v[pl.ds(k, 16)]
                m = iv < _SPLIT
                g = plsc.load_gather(slab_v, [zero16, iv], mask=m)
                plsc.store_scatter(orow_v, [zero16, lane + k], g, mask=m)

            copies = slab_dma(h, _SPLIT, _LN1)
            copies.append(
                pltpu.async_copy(
                    tailT_hbm.at[pl.ds(h, 1), :],
                    slab_v.at[:, pl.ds(_LN1, 128)],
                    sem,
                )
            )
            for c in copies:
                c.wait()

            @plsc.parallel_loop(0, BATCH, step=16, unroll=8)
            def step1(k):
                iv = idx_v[pl.ds(k, 16)]
                m = iv >= _SPLIT
                rel = jnp.where(
                    iv >= _AUX0, iv - (_AUX0 - _LN1), iv - _SPLIT
                )
                g = plsc.load_gather(slab_v, [zero16, rel], mask=m)
                plsc.store_scatter(orow_v, [zero16, lane + k], g, mask=m)

            pltpu.sync_copy(orow_v, outT_hbm.at[pl.ds(h, 1), :])

    return gather_kernel


_gather = _make_gather()


def kernel(problem_id, embedding_table):
    tail_t = embedding_table[_AUX0:].T
    out_t = _gather(problem_id, embedding_table.T, tail_t)
    return out_t.T


# speedup vs baseline: 1.3945x; 1.0041x over previous
import functools

import jax
import jax.numpy as jnp
from jax import lax
from jax.experimental import pallas as pl
from jax.experimental.pallas import tpu as pltpu
from jax.experimental.pallas import tpu_sc as plsc

NOP = 100000
HIDDEN_DIM = 64
BATCH = 16384

_info = plsc.get_sparse_core_info()
_NC, _NS = _info.num_cores, _info.num_subcores
_NW = _NC * _NS
_ROWS_PER_W = HIDDEN_DIM // _NW
_SPLIT = 50048
_ALIGNED_END = 99968
_AUX0 = NOP - 128
_LN1 = _ALIGNED_END - _SPLIT


def _make_gather():
    mesh = plsc.VectorSubcoreMesh(core_axis_name="c", subcore_axis_name="s")

    @functools.partial(
        pl.kernel,
        mesh=mesh,
        out_type=jax.ShapeDtypeStruct((HIDDEN_DIM, BATCH), jnp.float32),
        scratch_types=[
            pltpu.VMEM((BATCH,), jnp.int32),
            pltpu.VMEM((1, _SPLIT), jnp.float32),
            pltpu.VMEM((1, BATCH), jnp.float32),
            pltpu.SemaphoreType.DMA,
        ],
        compiler_params=pltpu.CompilerParams(
            needs_layout_passes=False, skip_device_barrier=True
        ),
    )
    def gather_kernel(
        idx_hbm, tT_hbm, tailT_hbm, outT_hbm, idx_v, slab_v, orow_v, sem
    ):
        wid = lax.axis_index("s") * _NC + lax.axis_index("c")
        lane = lax.iota(jnp.int32, 16)
        zero16 = jnp.zeros((16,), jnp.int32)

        def slab_dma(h, c0, ln):
            copies = []
            off = 0
            while off < ln:
                cl = min(12544, ln - off)
                copies.append(
                    pltpu.async_copy(
                        tT_hbm.at[pl.ds(h, 1), pl.ds(c0 + off, cl)],
                        slab_v.at[:, pl.ds(off, cl)],
                        sem,
                    )
                )
                off += cl
            return copies

        pltpu.sync_copy(idx_hbm, idx_v)

        def per_row(hp, _):
            h = _ROWS_PER_W * wid + hp

            for c in slab_dma(h, 0, _SPLIT):
                c.wait()

            @plsc.parallel_loop(0, BATCH, step=16, unroll=8)
            def step0(k):
                iv = idx_v[pl.ds(k, 16)]
                m = iv < _SPLIT
                g = plsc.load_gather(slab_v, [zero16, iv], mask=m)
                plsc.store_scatter(orow_v, [zero16, lane + k], g, mask=m)

            copies = slab_dma(h, _SPLIT, _LN1)
            copies.append(
                pltpu.async_copy(
                    tailT_hbm.at[pl.ds(h, 1), :],
                    slab_v.at[:, pl.ds(_LN1, 128)],
                    sem,
                )
            )
            for c in copies:
                c.wait()

            @plsc.parallel_loop(0, BATCH, step=16, unroll=8)
            def step1(k):
                iv = idx_v[pl.ds(k, 16)]
                m = iv >= _SPLIT
                rel = jnp.where(
                    iv >= _AUX0, iv - (_AUX0 - _LN1), iv - _SPLIT
                )
                g = plsc.load_gather(slab_v, [zero16, rel], mask=m)
                plsc.store_scatter(orow_v, [zero16, lane + k], g, mask=m)

            pltpu.sync_copy(orow_v, outT_hbm.at[pl.ds(h, 1), :])
            return ()

        lax.fori_loop(0, _ROWS_PER_W, per_row, ())

    return gather_kernel


_gather = _make_gather()


def kernel(problem_id, embedding_table):
    tail_t = embedding_table[_AUX0:].T
    out_t = _gather(problem_id, embedding_table.T, tail_t)
    return out_t.T
